# Initial kernel scaffold; baseline (speedup 1.0000x reference)
import jax
import jax.numpy as jnp
from jax.experimental import pallas as pl


def kernel(x, edge_index, edge_features, batch, W_node, W_edge, W1, W2, W3, W_out):
    raise NotImplementedError("write your pallas kernel here")


# speedup vs baseline: 2.0626x; 2.0626x over previous
import functools

import jax
import jax.numpy as jnp
from jax import lax
from jax.experimental import pallas as pl
from jax.experimental.pallas import tpu as pltpu
from jax.experimental.pallas import tpu_sc as plsc

N = 10000
E = 320000
IN = 128
H = 256
OUT = 128
L = 3
G = 64

NP = 10240
BN = 1024
BE = 1000
C = 80
NSUB = 16
EPT = E // NSUB
ROWS_PT = NP // NSUB


def _node_init_body(x_ref, wn_ref, wt_ref, h_ref, p_ref):
    h = x_ref[...] @ wn_ref[...]
    h_ref[...] = h
    p = h @ wt_ref[...]
    p_ref[0] = p[:, :128]
    p_ref[1] = p[:, 128:]


def _node_init(xp, w_node, w1t0, interpret=False):
    return pl.pallas_call(
        _node_init_body,
        grid=(NP // BN,),
        in_specs=[
            pl.BlockSpec((BN, IN), lambda i: (i, 0)),
            pl.BlockSpec((IN, H), lambda i: (0, 0)),
            pl.BlockSpec((H, H), lambda i: (0, 0)),
        ],
        out_specs=[
            pl.BlockSpec((BN, H), lambda i: (i, 0)),
            pl.BlockSpec((2, BN, 128), lambda i: (0, i, 0)),
        ],
        out_shape=[
            jax.ShapeDtypeStruct((NP, H), jnp.float32),
            jax.ShapeDtypeStruct((2, NP, 128), jnp.float32),
        ],
        interpret=interpret,
    )(xp, w_node, w1t0)


def _eterm_body(ef_ref, k_ref, out_ref):
    ef = ef_ref[...]
    for l in range(L):
        r = ef @ k_ref[l]
        out_ref[l, 0] = r[:, :128]
        out_ref[l, 1] = r[:, 128:]


def _eterm(edge_features, k, interpret=False):
    return pl.pallas_call(
        _eterm_body,
        grid=(E // BE,),
        in_specs=[
            pl.BlockSpec((BE, IN), lambda i: (i, 0)),
            pl.BlockSpec((L, IN, H), lambda i: (0, 0, 0)),
        ],
        out_specs=pl.BlockSpec((L, 2, BE, 128), lambda i: (0, 0, i, 0)),
        out_shape=jax.ShapeDtypeStruct((L, 2, E, 128), jnp.float32),
        interpret=interpret,
    )(edge_features, k)


def _update_body(h_ref, seg_ref, dp_ref, w23_ref, wt_ref, hn_ref, pn_ref):
    deg = jnp.sum(dp_ref[...], axis=0)
    inv = 1.0 / jnp.maximum(deg, 1.0)
    s0 = seg_ref[0] * inv[:, None]
    s1 = seg_ref[1] * inv[:, None]
    hn = h_ref[...] + s0 @ w23_ref[0] + s1 @ w23_ref[1]
    hn_ref[...] = hn
    p = hn @ wt_ref[...]
    pn_ref[0] = p[:, :128]
    pn_ref[1] = p[:, 128:]


def _update(h, seg, degpart, w23, w1t_next, interpret=False):
    return pl.pallas_call(
        _update_body,
        grid=(NP // BN,),
        in_specs=[
            pl.BlockSpec((BN, H), lambda i: (i, 0)),
            pl.BlockSpec((2, BN, 128), lambda i: (0, i, 0)),
            pl.BlockSpec((2 * NSUB, BN), lambda i: (0, i)),
            pl.BlockSpec((2, 128, H), lambda i: (0, 0, 0)),
            pl.BlockSpec((H, H), lambda i: (0, 0)),
        ],
        out_specs=[
            pl.BlockSpec((BN, H), lambda i: (i, 0)),
            pl.BlockSpec((2, BN, 128), lambda i: (0, i, 0)),
        ],
        out_shape=[
            jax.ShapeDtypeStruct((NP, H), jnp.float32),
            jax.ShapeDtypeStruct((2, NP, 128), jnp.float32),
        ],
        interpret=interpret,
    )(h, seg, degpart, w23, w1t_next)


def _final_body(h_ref, seg_ref, dp_ref, w23_ref, b_ref, wout_ref, out_ref,
                acc, cnt):
    i = pl.program_id(0)

    @pl.when(i == 0)
    def _():
        acc[...] = jnp.zeros_like(acc)
        cnt[...] = jnp.zeros_like(cnt)

    deg = jnp.sum(dp_ref[...], axis=0)
    inv = 1.0 / jnp.maximum(deg, 1.0)
    s0 = seg_ref[0] * inv[:, None]
    s1 = seg_ref[1] * inv[:, None]
    hn = h_ref[...] + s0 @ w23_ref[0] + s1 @ w23_ref[1]

    b = b_ref[0]
    gids = lax.broadcasted_iota(jnp.int32, (G, BN), 0)
    onehot = (b[None, :] == gids).astype(jnp.float32)
    acc[...] += onehot @ hn
    cnt[...] += jnp.broadcast_to(
        jnp.sum(onehot, axis=1, keepdims=True), (G, 128))

    @pl.when(i == NP // BN - 1)
    def _():
        invc = 1.0 / jnp.maximum(cnt[...], 1.0)
        pooled = acc[...] * jnp.concatenate([invc, invc], axis=1)
        out_ref[...] = pooled @ wout_ref[...]


def _final(h, seg, degpart, w23, batch_p, w_out, interpret=False):
    return pl.pallas_call(
        _final_body,
        grid=(NP // BN,),
        in_specs=[
            pl.BlockSpec((BN, H), lambda i: (i, 0)),
            pl.BlockSpec((2, BN, 128), lambda i: (0, i, 0)),
            pl.BlockSpec((2 * NSUB, BN), lambda i: (0, i)),
            pl.BlockSpec((2, 128, H), lambda i: (0, 0, 0)),
            pl.BlockSpec((1, BN), lambda i: (0, i)),
            pl.BlockSpec((H, OUT), lambda i: (0, 0)),
        ],
        out_specs=pl.BlockSpec((G, OUT), lambda i: (0, 0)),
        out_shape=jax.ShapeDtypeStruct((G, OUT), jnp.float32),
        scratch_shapes=[
            pltpu.VMEM((G, H), jnp.float32),
            pltpu.VMEM((G, 128), jnp.float32),
        ],
        interpret=interpret,
    )(h, seg, degpart, w23, batch_p, w_out)


@functools.lru_cache(maxsize=None)
def _mesh():
    return plsc.VectorSubcoreMesh(core_axis_name="c", subcore_axis_name="s")


@functools.lru_cache(maxsize=None)
def _make_deg_kernel():
    return functools.partial(
        pl.kernel,
        out_type=jax.ShapeDtypeStruct((2 * NSUB, NP), jnp.float32),
        mesh=_mesh(),
        scratch_types=[
            pltpu.VMEM((NP,), jnp.float32),
            pltpu.VMEM((C,), jnp.int32),
        ],
        compiler_params=pltpu.CompilerParams(needs_layout_passes=False),
    )(_deg_body)


def _deg_body(dst_hbm, out_hbm, hist_v, idx_v):
    cc = lax.axis_index("c")
    ss = lax.axis_index("s")
    wid = ss * 2 + cc
    zeros16 = jnp.zeros((16,), jnp.float32)
    ones16 = jnp.ones((16,), jnp.float32)

    def zbody(j, _):
        hist_v[pl.ds(j * 16, 16)] = zeros16
        return 0

    lax.fori_loop(0, NP // 16, zbody, 0)

    ept = E // (2 * NSUB)

    def chbody(ch, _):
        base = wid * ept + ch * C
        pltpu.sync_copy(dst_hbm.at[pl.ds(base, C)], idx_v)
        for j in range(C // 16):
            idx = idx_v[pl.ds(j * 16, 16)]
            plsc.addupdate_scatter(hist_v, [idx], ones16)
        return 0

    lax.fori_loop(0, ept // C, chbody, 0)
    pltpu.sync_copy(hist_v, out_hbm.at[wid])


@functools.lru_cache(maxsize=None)
def _make_edge_kernel(l):
    @functools.partial(
        pl.kernel,
        out_type=jax.ShapeDtypeStruct((2, NP, 128), jnp.float32),
        mesh=_mesh(),
        scratch_types=[
            pltpu.VMEM((C,), jnp.int32),
            pltpu.VMEM((C,), jnp.int32),
            pltpu.VMEM((C, 128), jnp.float32),
            pltpu.VMEM((C, 128), jnp.float32),
            pltpu.VMEM_SHARED((NP, 128), jnp.float32),
            pltpu.SemaphoreType.DMA,
        ],
        compiler_params=pltpu.CompilerParams(needs_layout_passes=False),
    )
    def _edge_kernel(p_hbm, et_hbm, src_hbm, dst_hbm, seg_hbm,
                     src_v, dst_v, rows_v, et_v, acc_sh, sem):
        cc = lax.axis_index("c")
        ss = lax.axis_index("s")
        zeros16 = jnp.zeros((16,), jnp.float32)

        def zrow(r, _):
            for j in range(8):
                rows_v[r, pl.ds(j * 16, 16)] = zeros16
            return 0

        lax.fori_loop(0, C, zrow, 0)
        for k in range(ROWS_PT // C):
            pltpu.sync_copy(rows_v, acc_sh.at[pl.ds(ss * ROWS_PT + k * C, C)])
        plsc.subcore_barrier()

        coff = cc * NP

        def chbody(ch, _):
            e0 = ss * EPT + ch * C
            pltpu.sync_copy(src_hbm.at[pl.ds(e0, C)], src_v)
            pltpu.sync_copy(dst_hbm.at[pl.ds(e0, C)], dst_v)
            for j in range(C // 16):
                src_v[pl.ds(j * 16, 16)] = src_v[pl.ds(j * 16, 16)] + coff
            pltpu.async_copy(p_hbm.at[src_v], rows_v, sem).wait()
            pltpu.sync_copy(et_hbm.at[l].at[cc].at[pl.ds(e0, C)], et_v)

            def rbody(r, _):
                for j in range(8):
                    sl = pl.ds(j * 16, 16)
                    v = rows_v[r, sl] + et_v[r, sl]
                    rows_v[r, sl] = jnp.maximum(v, 0.0)
                return 0

            lax.fori_loop(0, C, rbody, 0)
            pltpu.sync_copy(rows_v, acc_sh.at[dst_v], add=True)
            return 0

        lax.fori_loop(0, EPT // C, chbody, 0)
        plsc.subcore_barrier()

        for k in range(ROWS_PT // C):
            base = ss * ROWS_PT + k * C
            pltpu.sync_copy(acc_sh.at[pl.ds(base, C)], rows_v)
            pltpu.sync_copy(rows_v, seg_hbm.at[cc].at[pl.ds(base, C)])

    return _edge_kernel


def kernel(x, edge_index, edge_features, batch, W_node, W_edge, W1, W2, W3,
           W_out):
    src = edge_index[0]
    dst = edge_index[1]
    xp = jnp.pad(x, ((0, NP - N), (0, 0)))
    batch_p = jnp.pad(batch, (0, NP - N), constant_values=G).reshape(1, NP)

    w1t = W1[:, :H, :]
    w1b = W1[:, H:, :]
    k = jnp.einsum("ih,lho->lio", W_edge, w1b)
    w23 = jnp.einsum("lhk,lko->lho", W2, W3).reshape(L, 2, 128, H)

    h, p = _node_init(xp, W_node, w1t[0])
    et = _eterm(edge_features, k)
    degpart = _make_deg_kernel()(dst)
    src = lax.optimization_barrier((src, degpart))[0]

    for l in range(L):
        seg = _make_edge_kernel(l)(p.reshape(2 * NP, 128), et, src, dst)
        if l < L - 1:
            h, p = _update(h, seg, degpart, w23[l], w1t[l + 1])
        else:
            out = _final(h, seg, degpart, w23[l], batch_p, W_out)
    return out


# speedup vs baseline: 3.9075x; 1.8944x over previous
import functools

import jax
import jax.numpy as jnp
from jax import lax
from jax.experimental import pallas as pl
from jax.experimental.pallas import tpu as pltpu
from jax.experimental.pallas import tpu_sc as plsc

N = 10000
E = 320000
IN = 128
H = 256
OUT = 128
L = 3
G = 64

NP = 10240
BN = 1024
BE = 1000
C = 80
NSUB = 16
EPT = E // NSUB
ROWS_PT = NP // NSUB


def _node_init_body(x_ref, wn_ref, wt_ref, h_ref, p_ref):
    h = x_ref[...] @ wn_ref[...]
    h_ref[...] = h
    p = h @ wt_ref[...]
    p_ref[0] = p[:, :128]
    p_ref[1] = p[:, 128:]


def _node_init(xp, w_node, w1t0, interpret=False):
    return pl.pallas_call(
        _node_init_body,
        grid=(NP // BN,),
        in_specs=[
            pl.BlockSpec((BN, IN), lambda i: (i, 0)),
            pl.BlockSpec((IN, H), lambda i: (0, 0)),
            pl.BlockSpec((H, H), lambda i: (0, 0)),
        ],
        out_specs=[
            pl.BlockSpec((BN, H), lambda i: (i, 0)),
            pl.BlockSpec((2, BN, 128), lambda i: (0, i, 0)),
        ],
        out_shape=[
            jax.ShapeDtypeStruct((NP, H), jnp.float32),
            jax.ShapeDtypeStruct((2, NP, 128), jnp.float32),
        ],
        interpret=interpret,
    )(xp, w_node, w1t0)


def _eterm_body(ef_ref, k_ref, out_ref):
    ef = ef_ref[...]
    for l in range(L):
        r = ef @ k_ref[l]
        out_ref[l, 0] = r[:, :128]
        out_ref[l, 1] = r[:, 128:]


def _eterm(edge_features, k, interpret=False):
    return pl.pallas_call(
        _eterm_body,
        grid=(E // BE,),
        in_specs=[
            pl.BlockSpec((BE, IN), lambda i: (i, 0)),
            pl.BlockSpec((L, IN, H), lambda i: (0, 0, 0)),
        ],
        out_specs=pl.BlockSpec((L, 2, BE, 128), lambda i: (0, 0, i, 0)),
        out_shape=jax.ShapeDtypeStruct((L, 2, E, 128), jnp.float32),
        interpret=interpret,
    )(edge_features, k)


def _update_body(h_ref, seg_ref, dp_ref, w23_ref, wt_ref, hn_ref, pn_ref):
    deg = jnp.sum(dp_ref[...], axis=0)
    inv = 1.0 / jnp.maximum(deg, 1.0)
    s0 = seg_ref[0] * inv[:, None]
    s1 = seg_ref[1] * inv[:, None]
    hn = h_ref[...] + s0 @ w23_ref[0] + s1 @ w23_ref[1]
    hn_ref[...] = hn
    p = hn @ wt_ref[...]
    pn_ref[0] = p[:, :128]
    pn_ref[1] = p[:, 128:]


def _update(h, seg, degpart, w23, w1t_next, interpret=False):
    return pl.pallas_call(
        _update_body,
        grid=(NP // BN,),
        in_specs=[
            pl.BlockSpec((BN, H), lambda i: (i, 0)),
            pl.BlockSpec((2, BN, 128), lambda i: (0, i, 0)),
            pl.BlockSpec((NSUB, BN), lambda i: (0, i)),
            pl.BlockSpec((2, 128, H), lambda i: (0, 0, 0)),
            pl.BlockSpec((H, H), lambda i: (0, 0)),
        ],
        out_specs=[
            pl.BlockSpec((BN, H), lambda i: (i, 0)),
            pl.BlockSpec((2, BN, 128), lambda i: (0, i, 0)),
        ],
        out_shape=[
            jax.ShapeDtypeStruct((NP, H), jnp.float32),
            jax.ShapeDtypeStruct((2, NP, 128), jnp.float32),
        ],
        interpret=interpret,
    )(h, seg, degpart, w23, w1t_next)


def _final_body(h_ref, seg_ref, dp_ref, w23_ref, b_ref, wout_ref, out_ref,
                acc, cnt):
    i = pl.program_id(0)

    @pl.when(i == 0)
    def _():
        acc[...] = jnp.zeros_like(acc)
        cnt[...] = jnp.zeros_like(cnt)

    deg = jnp.sum(dp_ref[...], axis=0)
    inv = 1.0 / jnp.maximum(deg, 1.0)
    s0 = seg_ref[0] * inv[:, None]
    s1 = seg_ref[1] * inv[:, None]
    hn = h_ref[...] + s0 @ w23_ref[0] + s1 @ w23_ref[1]

    b = b_ref[0]
    gids = lax.broadcasted_iota(jnp.int32, (G, BN), 0)
    onehot = (b[None, :] == gids).astype(jnp.float32)
    acc[...] += onehot @ hn
    cnt[...] += jnp.broadcast_to(
        jnp.sum(onehot, axis=1, keepdims=True), (G, 128))

    @pl.when(i == NP // BN - 1)
    def _():
        invc = 1.0 / jnp.maximum(cnt[...], 1.0)
        pooled = acc[...] * jnp.concatenate([invc, invc], axis=1)
        out_ref[...] = pooled @ wout_ref[...]


def _final(h, seg, degpart, w23, batch_p, w_out, interpret=False):
    return pl.pallas_call(
        _final_body,
        grid=(NP // BN,),
        in_specs=[
            pl.BlockSpec((BN, H), lambda i: (i, 0)),
            pl.BlockSpec((2, BN, 128), lambda i: (0, i, 0)),
            pl.BlockSpec((NSUB, BN), lambda i: (0, i)),
            pl.BlockSpec((2, 128, H), lambda i: (0, 0, 0)),
            pl.BlockSpec((1, BN), lambda i: (0, i)),
            pl.BlockSpec((H, OUT), lambda i: (0, 0)),
        ],
        out_specs=pl.BlockSpec((G, OUT), lambda i: (0, 0)),
        out_shape=jax.ShapeDtypeStruct((G, OUT), jnp.float32),
        scratch_shapes=[
            pltpu.VMEM((G, H), jnp.float32),
            pltpu.VMEM((G, 128), jnp.float32),
        ],
        interpret=interpret,
    )(h, seg, degpart, w23, batch_p, w_out)


@functools.lru_cache(maxsize=None)
def _mesh():
    return plsc.VectorSubcoreMesh(core_axis_name="c", subcore_axis_name="s")


NCH = EPT // C


@functools.lru_cache(maxsize=None)
def _make_edge_kernel(l, with_deg):
    seg_type = jax.ShapeDtypeStruct((2, NP, 128), jnp.float32)
    if with_deg:
        out_type = [seg_type, jax.ShapeDtypeStruct((NSUB, NP), jnp.float32)]
    else:
        out_type = seg_type
    scratch = [
        pltpu.VMEM((2, C), jnp.int32),
        pltpu.VMEM((2, C), jnp.int32),
        pltpu.VMEM((2, C), jnp.int32),
        pltpu.VMEM((2, C), jnp.int32),
        pltpu.VMEM((2, C, 128), jnp.float32),
        pltpu.VMEM((C, 128), jnp.float32),
        pltpu.VMEM_SHARED((NP, 128), jnp.float32),
    ]
    if with_deg:
        scratch.append(pltpu.VMEM((NP,), jnp.float32))
    scratch += [pltpu.SemaphoreType.DMA] * 7

    def _edge_body(*refs):
        p_hbm, et_hbm, src_hbm, dst_hbm, seg_hbm = refs[:5]
        refs = refs[5:]
        if with_deg:
            deg_hbm = refs[0]
            refs = refs[1:]
        src_v, dst_v, gidx_v, sidx_v, rows_v, et_v, acc_sh = refs[:7]
        refs = refs[7:]
        if with_deg:
            hist_v = refs[0]
            refs = refs[1:]
        isem = refs[0:2]
        esem = refs[2]
        gsem = refs[3:5]
        ssem = refs[5:7]

        cc = lax.axis_index("c")
        ss = lax.axis_index("s")
        zeros16 = jnp.zeros((16,), jnp.float32)
        ones16 = jnp.ones((16,), jnp.float32)
        coff = cc * NP
        e_base = ss * EPT
        et_l = et_hbm.at[l].at[cc]

        rz = rows_v.at[0]

        def zrow(r, _):
            for j in range(8):
                rz[r, pl.ds(j * 16, 16)] = zeros16
            return 0

        lax.fori_loop(0, C, zrow, 0)
        for k in range(ROWS_PT // C):
            pltpu.sync_copy(rz, acc_sh.at[pl.ds(ss * ROWS_PT + k * C, C)])
        if with_deg:
            def zhist(j, _):
                hist_v[pl.ds(j * 16, 16)] = zeros16
                return 0

            lax.fori_loop(0, NP // 16, zhist, 0)

        def prep(b, kbase):
            for j in range(C // 16):
                sl = pl.ds(j * 16, 16)
                d = dst_v[b, sl]
                sidx_v[b, sl] = d
                gidx_v[b, sl] = src_v[b, sl] + coff
                if with_deg:
                    @pl.when(cc == 0)
                    def _():
                        plsc.addupdate_scatter(hist_v, [d], ones16)

        def start_idx(b, kbase):
            pltpu.async_copy(src_hbm.at[pl.ds(kbase, C)], src_v.at[b],
                             isem[b])
            pltpu.async_copy(dst_hbm.at[pl.ds(kbase, C)], dst_v.at[b],
                             isem[b])

        def wait_idx(b, kbase):
            pltpu.make_async_copy(src_hbm.at[pl.ds(kbase, C)], src_v.at[b],
                                  isem[b]).wait()
            pltpu.make_async_copy(dst_hbm.at[pl.ds(kbase, C)], dst_v.at[b],
                                  isem[b]).wait()

        def start_gather(b):
            pltpu.async_copy(p_hbm.at[gidx_v.at[b]], rows_v.at[b], gsem[b])

        def wait_gather(b):
            pltpu.make_async_copy(p_hbm.at[gidx_v.at[b]], rows_v.at[b],
                                  gsem[b]).wait()

        def start_et(kbase):
            pltpu.async_copy(et_l.at[pl.ds(kbase, C)], et_v, esem)

        def wait_et(kbase):
            pltpu.make_async_copy(et_l.at[pl.ds(kbase, C)], et_v,
                                  esem).wait()

        def start_scatter(b):
            pltpu.async_copy(rows_v.at[b], acc_sh.at[sidx_v.at[b]], ssem[b],
                             add=True)

        def wait_scatter(b):
            pltpu.make_async_copy(rows_v.at[b], acc_sh.at[sidx_v.at[b]],
                                  ssem[b]).wait()

        start_idx(0, e_base)
        start_et(e_base)
        start_idx(1, e_base + C)
        wait_idx(0, e_base)
        prep(0, e_base)
        start_gather(0)
        plsc.subcore_barrier()

        def chbody(ch, _):
            for b in (0, 1):
                nb = 1 - b
                k = 2 * ch + b
                kb = e_base + k * C

                if b == 0:
                    @pl.when(ch > 0)
                    def _():
                        wait_scatter(nb)
                else:
                    wait_scatter(nb)

                def launch_next():
                    wait_idx(nb, kb + C)
                    prep(nb, kb + C)
                    start_gather(nb)

                if b == 0:
                    launch_next()
                else:
                    @pl.when(ch < NCH // 2 - 1)
                    def _():
                        launch_next()

                @pl.when(ch < NCH // 2 - 1)
                def _():
                    start_idx(b, kb + 2 * C)

                wait_et(kb)
                wait_gather(b)
                rv = rows_v.at[b]

                def rbody(r, _):
                    for j in range(8):
                        sl = pl.ds(j * 16, 16)
                        rv[r, sl] = jnp.maximum(rv[r, sl] + et_v[r, sl], 0.0)
                    return 0

                lax.fori_loop(0, C, rbody, 0)

                if b == 0:
                    start_et(kb + C)
                else:
                    @pl.when(ch < NCH // 2 - 1)
                    def _():
                        start_et(kb + C)

                start_scatter(b)
            return 0

        lax.fori_loop(0, NCH // 2, chbody, 0)
        wait_scatter(1)
        plsc.subcore_barrier()

        rs = rows_v.at[0]
        for k in range(ROWS_PT // C):
            base = ss * ROWS_PT + k * C
            pltpu.sync_copy(acc_sh.at[pl.ds(base, C)], rs)
            pltpu.sync_copy(rs, seg_hbm.at[cc].at[pl.ds(base, C)])
        if with_deg:
            @pl.when(cc == 0)
            def _():
                pltpu.sync_copy(hist_v, deg_hbm.at[ss])

    return functools.partial(
        pl.kernel,
        out_type=out_type,
        mesh=_mesh(),
        scratch_types=scratch,
        compiler_params=pltpu.CompilerParams(needs_layout_passes=False),
    )(_edge_body)


def kernel(x, edge_index, edge_features, batch, W_node, W_edge, W1, W2, W3,
           W_out):
    src = edge_index[0]
    dst = edge_index[1]
    xp = jnp.pad(x, ((0, NP - N), (0, 0)))
    batch_p = jnp.pad(batch, (0, NP - N), constant_values=G).reshape(1, NP)

    w1t = W1[:, :H, :]
    w1b = W1[:, H:, :]
    k = jnp.einsum("ih,lho->lio", W_edge, w1b)
    w23 = jnp.einsum("lhk,lko->lho", W2, W3).reshape(L, 2, 128, H)

    h, p = _node_init(xp, W_node, w1t[0])
    et = _eterm(edge_features, k)

    for l in range(L):
        if l == 0:
            seg, degpart = _make_edge_kernel(0, True)(
                p.reshape(2 * NP, 128), et, src, dst)
        else:
            seg = _make_edge_kernel(l, False)(
                p.reshape(2 * NP, 128), et, src, dst)
        if l < L - 1:
            h, p = _update(h, seg, degpart, w23[l], w1t[l + 1])
        else:
            out = _final(h, seg, degpart, w23[l], batch_p, W_out)
    return out


# speedup vs baseline: 3.9853x; 1.0199x over previous
import functools

import jax
import jax.numpy as jnp
from jax import lax
from jax.experimental import pallas as pl
from jax.experimental.pallas import tpu as pltpu
from jax.experimental.pallas import tpu_sc as plsc

N = 10000
E = 320000
IN = 128
H = 256
OUT = 128
L = 3
G = 64

NP = 10240
BN = 1024
BE = 1000
C = 80
NSUB = 16
EPT = E // NSUB
ROWS_PT = NP // NSUB


def _node_init_body(x_ref, wn_ref, wt_ref, h_ref, p_ref):
    h = x_ref[...] @ wn_ref[...]
    h_ref[...] = h
    p = h @ wt_ref[...]
    p_ref[0] = p[:, :128]
    p_ref[1] = p[:, 128:]


def _node_init(xp, w_node, w1t0, interpret=False):
    return pl.pallas_call(
        _node_init_body,
        grid=(NP // BN,),
        in_specs=[
            pl.BlockSpec((BN, IN), lambda i: (i, 0)),
            pl.BlockSpec((IN, H), lambda i: (0, 0)),
            pl.BlockSpec((H, H), lambda i: (0, 0)),
        ],
        out_specs=[
            pl.BlockSpec((BN, H), lambda i: (i, 0)),
            pl.BlockSpec((2, BN, 128), lambda i: (0, i, 0)),
        ],
        out_shape=[
            jax.ShapeDtypeStruct((NP, H), jnp.float32),
            jax.ShapeDtypeStruct((2, NP, 128), jnp.float32),
        ],
        interpret=interpret,
    )(xp, w_node, w1t0)


def _make_eterm_body(nl):
    def _eterm_body(ef_ref, k_ref, out_ref):
        ef = ef_ref[...]
        for l in range(nl):
            r = ef @ k_ref[l]
            out_ref[l, 0] = r[:, :128]
            out_ref[l, 1] = r[:, 128:]
    return _eterm_body


def _eterm(edge_features, k, interpret=False):
    nl = k.shape[0]
    return pl.pallas_call(
        _make_eterm_body(nl),
        grid=(E // BE,),
        in_specs=[
            pl.BlockSpec((BE, IN), lambda i: (i, 0)),
            pl.BlockSpec((nl, IN, H), lambda i: (0, 0, 0)),
        ],
        out_specs=pl.BlockSpec((nl, 2, BE, 128), lambda i: (0, 0, i, 0)),
        out_shape=jax.ShapeDtypeStruct((nl, 2, E, 128), jnp.float32),
        interpret=interpret,
    )(edge_features, k)


def _update_body(h_ref, seg_ref, dp_ref, w23_ref, wt_ref, hn_ref, pn_ref):
    deg = jnp.sum(dp_ref[...], axis=0)
    inv = 1.0 / jnp.maximum(deg, 1.0)
    s0 = seg_ref[0] * inv[:, None]
    s1 = seg_ref[1] * inv[:, None]
    hn = h_ref[...] + s0 @ w23_ref[0] + s1 @ w23_ref[1]
    hn_ref[...] = hn
    p = hn @ wt_ref[...]
    pn_ref[0] = p[:, :128]
    pn_ref[1] = p[:, 128:]


def _update(h, seg, degpart, w23, w1t_next, interpret=False):
    return pl.pallas_call(
        _update_body,
        grid=(NP // BN,),
        in_specs=[
            pl.BlockSpec((BN, H), lambda i: (i, 0)),
            pl.BlockSpec((2, BN, 128), lambda i: (0, i, 0)),
            pl.BlockSpec((NSUB, BN), lambda i: (0, i)),
            pl.BlockSpec((2, 128, H), lambda i: (0, 0, 0)),
            pl.BlockSpec((H, H), lambda i: (0, 0)),
        ],
        out_specs=[
            pl.BlockSpec((BN, H), lambda i: (i, 0)),
            pl.BlockSpec((2, BN, 128), lambda i: (0, i, 0)),
        ],
        out_shape=[
            jax.ShapeDtypeStruct((NP, H), jnp.float32),
            jax.ShapeDtypeStruct((2, NP, 128), jnp.float32),
        ],
        interpret=interpret,
    )(h, seg, degpart, w23, w1t_next)


def _final_body(h_ref, seg_ref, dp_ref, w23_ref, b_ref, wout_ref, out_ref,
                acc, cnt):
    i = pl.program_id(0)

    @pl.when(i == 0)
    def _():
        acc[...] = jnp.zeros_like(acc)
        cnt[...] = jnp.zeros_like(cnt)

    deg = jnp.sum(dp_ref[...], axis=0)
    inv = 1.0 / jnp.maximum(deg, 1.0)
    s0 = seg_ref[0] * inv[:, None]
    s1 = seg_ref[1] * inv[:, None]
    hn = h_ref[...] + s0 @ w23_ref[0] + s1 @ w23_ref[1]

    b = b_ref[0]
    gids = lax.broadcasted_iota(jnp.int32, (G, BN), 0)
    onehot = (b[None, :] == gids).astype(jnp.float32)
    acc[...] += onehot @ hn
    cnt[...] += jnp.broadcast_to(
        jnp.sum(onehot, axis=1, keepdims=True), (G, 128))

    @pl.when(i == NP // BN - 1)
    def _():
        invc = 1.0 / jnp.maximum(cnt[...], 1.0)
        pooled = acc[...] * jnp.concatenate([invc, invc], axis=1)
        out_ref[...] = pooled @ wout_ref[...]


def _final(h, seg, degpart, w23, batch_p, w_out, interpret=False):
    return pl.pallas_call(
        _final_body,
        grid=(NP // BN,),
        in_specs=[
            pl.BlockSpec((BN, H), lambda i: (i, 0)),
            pl.BlockSpec((2, BN, 128), lambda i: (0, i, 0)),
            pl.BlockSpec((NSUB, BN), lambda i: (0, i)),
            pl.BlockSpec((2, 128, H), lambda i: (0, 0, 0)),
            pl.BlockSpec((1, BN), lambda i: (0, i)),
            pl.BlockSpec((H, OUT), lambda i: (0, 0)),
        ],
        out_specs=pl.BlockSpec((G, OUT), lambda i: (0, 0)),
        out_shape=jax.ShapeDtypeStruct((G, OUT), jnp.float32),
        scratch_shapes=[
            pltpu.VMEM((G, H), jnp.float32),
            pltpu.VMEM((G, 128), jnp.float32),
        ],
        interpret=interpret,
    )(h, seg, degpart, w23, batch_p, w_out)


@functools.lru_cache(maxsize=None)
def _mesh():
    return plsc.VectorSubcoreMesh(core_axis_name="c", subcore_axis_name="s")


NCH = EPT // C


@functools.lru_cache(maxsize=None)
def _make_edge_kernel(l, with_deg):
    seg_type = jax.ShapeDtypeStruct((2, NP, 128), jnp.float32)
    if with_deg:
        out_type = [seg_type, jax.ShapeDtypeStruct((NSUB, NP), jnp.float32)]
    else:
        out_type = seg_type
    scratch = [
        pltpu.VMEM((2, C), jnp.int32),
        pltpu.VMEM((2, C), jnp.int32),
        pltpu.VMEM((2, C), jnp.int32),
        pltpu.VMEM((2, C), jnp.int32),
        pltpu.VMEM((2, C, 128), jnp.float32),
        pltpu.VMEM((C, 128), jnp.float32),
        pltpu.VMEM_SHARED((NP, 128), jnp.float32),
    ]
    if with_deg:
        scratch.append(pltpu.VMEM((NP,), jnp.float32))
    scratch += [pltpu.SemaphoreType.DMA] * 7

    def _edge_body(*refs):
        p_hbm, et_hbm, src_hbm, dst_hbm, seg_hbm = refs[:5]
        refs = refs[5:]
        if with_deg:
            deg_hbm = refs[0]
            refs = refs[1:]
        src_v, dst_v, gidx_v, sidx_v, rows_v, et_v, acc_sh = refs[:7]
        refs = refs[7:]
        if with_deg:
            hist_v = refs[0]
            refs = refs[1:]
        isem = refs[0:2]
        esem = refs[2]
        gsem = refs[3:5]
        ssem = refs[5:7]

        cc = lax.axis_index("c")
        ss = lax.axis_index("s")
        zeros16 = jnp.zeros((16,), jnp.float32)
        ones16 = jnp.ones((16,), jnp.float32)
        coff = cc * NP
        e_base = ss * EPT
        et_l = et_hbm.at[l].at[cc]

        rz = rows_v.at[0]

        def zrow(r, _):
            for j in range(8):
                rz[r, pl.ds(j * 16, 16)] = zeros16
            return 0

        lax.fori_loop(0, C, zrow, 0)
        for k in range(ROWS_PT // C):
            pltpu.sync_copy(rz, acc_sh.at[pl.ds(ss * ROWS_PT + k * C, C)])
        if with_deg:
            def zhist(j, _):
                hist_v[pl.ds(j * 16, 16)] = zeros16
                return 0

            lax.fori_loop(0, NP // 16, zhist, 0)

        def prep(b, kbase):
            for j in range(C // 16):
                sl = pl.ds(j * 16, 16)
                d = dst_v[b, sl]
                sidx_v[b, sl] = d
                gidx_v[b, sl] = src_v[b, sl] + coff
                if with_deg:
                    @pl.when(cc == 0)
                    def _():
                        plsc.addupdate_scatter(hist_v, [d], ones16)

        def start_idx(b, kbase):
            pltpu.async_copy(src_hbm.at[pl.ds(kbase, C)], src_v.at[b],
                             isem[b])
            pltpu.async_copy(dst_hbm.at[pl.ds(kbase, C)], dst_v.at[b],
                             isem[b])

        def wait_idx(b, kbase):
            pltpu.make_async_copy(src_hbm.at[pl.ds(kbase, C)], src_v.at[b],
                                  isem[b]).wait()
            pltpu.make_async_copy(dst_hbm.at[pl.ds(kbase, C)], dst_v.at[b],
                                  isem[b]).wait()

        def start_gather(b):
            pltpu.async_copy(p_hbm.at[gidx_v.at[b]], rows_v.at[b], gsem[b])

        def wait_gather(b):
            pltpu.make_async_copy(p_hbm.at[gidx_v.at[b]], rows_v.at[b],
                                  gsem[b]).wait()

        def start_et(kbase):
            pltpu.async_copy(et_l.at[pl.ds(kbase, C)], et_v, esem)

        def wait_et(kbase):
            pltpu.make_async_copy(et_l.at[pl.ds(kbase, C)], et_v,
                                  esem).wait()

        def start_scatter(b):
            pltpu.async_copy(rows_v.at[b], acc_sh.at[sidx_v.at[b]], ssem[b],
                             add=True)

        def wait_scatter(b):
            pltpu.make_async_copy(rows_v.at[b], acc_sh.at[sidx_v.at[b]],
                                  ssem[b]).wait()

        start_idx(0, e_base)
        start_et(e_base)
        start_idx(1, e_base + C)
        wait_idx(0, e_base)
        prep(0, e_base)
        start_gather(0)
        plsc.subcore_barrier()

        def chbody(ch, _):
            for b in (0, 1):
                nb = 1 - b
                k = 2 * ch + b
                kb = e_base + k * C

                if b == 0:
                    @pl.when(ch > 0)
                    def _():
                        wait_scatter(nb)
                else:
                    wait_scatter(nb)

                def launch_next():
                    wait_idx(nb, kb + C)
                    prep(nb, kb + C)
                    start_gather(nb)

                if b == 0:
                    launch_next()
                else:
                    @pl.when(ch < NCH // 2 - 1)
                    def _():
                        launch_next()

                @pl.when(ch < NCH // 2 - 1)
                def _():
                    start_idx(b, kb + 2 * C)

                wait_et(kb)
                wait_gather(b)
                rv = rows_v.at[b]

                def rbody(r, _):
                    for j in range(8):
                        sl = pl.ds(j * 16, 16)
                        rv[r, sl] = jnp.maximum(rv[r, sl] + et_v[r, sl], 0.0)
                    return 0

                lax.fori_loop(0, C, rbody, 0)

                if b == 0:
                    start_et(kb + C)
                else:
                    @pl.when(ch < NCH // 2 - 1)
                    def _():
                        start_et(kb + C)

                start_scatter(b)
            return 0

        lax.fori_loop(0, NCH // 2, chbody, 0)
        wait_scatter(1)
        plsc.subcore_barrier()

        rs = rows_v.at[0]
        for k in range(ROWS_PT // C):
            base = ss * ROWS_PT + k * C
            pltpu.sync_copy(acc_sh.at[pl.ds(base, C)], rs)
            pltpu.sync_copy(rs, seg_hbm.at[cc].at[pl.ds(base, C)])
        if with_deg:
            @pl.when(cc == 0)
            def _():
                pltpu.sync_copy(hist_v, deg_hbm.at[ss])

    return functools.partial(
        pl.kernel,
        out_type=out_type,
        mesh=_mesh(),
        scratch_types=scratch,
        compiler_params=pltpu.CompilerParams(needs_layout_passes=False),
    )(_edge_body)


def kernel(x, edge_index, edge_features, batch, W_node, W_edge, W1, W2, W3,
           W_out):
    src = edge_index[0]
    dst = edge_index[1]
    xp = jnp.pad(x, ((0, NP - N), (0, 0)))
    batch_p = jnp.pad(batch, (0, NP - N), constant_values=G).reshape(1, NP)

    w1t = W1[:, :H, :]
    w1b = W1[:, H:, :]
    k = jnp.einsum("ih,lho->lio", W_edge, w1b)
    w23 = jnp.einsum("lhk,lko->lho", W2, W3).reshape(L, 2, 128, H)

    h, p = _node_init(xp, W_node, w1t[0])
    et0 = _eterm(edge_features, k[:1])
    et12 = _eterm(edge_features, k[1:])

    for l in range(L):
        if l == 0:
            seg, degpart = _make_edge_kernel(0, True)(
                p.reshape(2 * NP, 128), et0, src, dst)
        else:
            seg = _make_edge_kernel(l - 1, False)(
                p.reshape(2 * NP, 128), et12, src, dst)
        if l < L - 1:
            h, p = _update(h, seg, degpart, w23[l], w1t[l + 1])
        else:
            out = _final(h, seg, degpart, w23[l], batch_p, W_out)
    return out


# speedup vs baseline: 4.2054x; 1.0552x over previous
import functools

import jax
import jax.numpy as jnp
from jax import lax
from jax.experimental import pallas as pl
from jax.experimental.pallas import tpu as pltpu
from jax.experimental.pallas import tpu_sc as plsc

N = 10000
E = 320000
IN = 128
H = 256
OUT = 128
L = 3
G = 64

NP = 10240
BN = 1024
BE = 1000
C = 80
NSUB = 16
EPT = E // NSUB
ROWS_PT = NP // NSUB


def _node_init_body(x_ref, wn_ref, wt_ref, h_ref, p_ref):
    h = x_ref[...] @ wn_ref[...]
    h_ref[...] = h
    p = h @ wt_ref[...]
    p_ref[0] = p[:, :128]
    p_ref[1] = p[:, 128:]


def _node_init(xp, w_node, w1t0, interpret=False):
    return pl.pallas_call(
        _node_init_body,
        grid=(NP // BN,),
        in_specs=[
            pl.BlockSpec((BN, IN), lambda i: (i, 0)),
            pl.BlockSpec((IN, H), lambda i: (0, 0)),
            pl.BlockSpec((H, H), lambda i: (0, 0)),
        ],
        out_specs=[
            pl.BlockSpec((BN, H), lambda i: (i, 0)),
            pl.BlockSpec((2, BN, 128), lambda i: (0, i, 0)),
        ],
        out_shape=[
            jax.ShapeDtypeStruct((NP, H), jnp.float32),
            jax.ShapeDtypeStruct((2, NP, 128), jnp.float32),
        ],
        interpret=interpret,
    )(xp, w_node, w1t0)


def _pack_bf16_pairs(m):
    lo = jax.lax.bitcast_convert_type(
        m[:, :64].astype(jnp.bfloat16), jnp.uint16).astype(jnp.uint32)
    hi = jax.lax.bitcast_convert_type(
        m[:, 64:].astype(jnp.bfloat16), jnp.uint16).astype(jnp.uint32)
    return lo | (hi << 16)


def _make_eterm_body(nl):
    def _eterm_body(ef_ref, k_ref, out_ref):
        ef = ef_ref[...]
        for l in range(nl):
            r = ef @ k_ref[l]
            out_ref[l, 0] = _pack_bf16_pairs(r[:, :128])
            out_ref[l, 1] = _pack_bf16_pairs(r[:, 128:])
    return _eterm_body


def _eterm(edge_features, k, interpret=False):
    nl = k.shape[0]
    return pl.pallas_call(
        _make_eterm_body(nl),
        grid=(E // BE,),
        in_specs=[
            pl.BlockSpec((BE, IN), lambda i: (i, 0)),
            pl.BlockSpec((nl, IN, H), lambda i: (0, 0, 0)),
        ],
        out_specs=pl.BlockSpec((nl, 2, BE, 64), lambda i: (0, 0, i, 0)),
        out_shape=jax.ShapeDtypeStruct((nl, 2, E, 64), jnp.uint32),
        interpret=interpret,
    )(edge_features, k)


def _update_body(h_ref, seg_ref, dp_ref, w23_ref, wt_ref, hn_ref, pn_ref):
    deg = jnp.sum(dp_ref[...], axis=0)
    inv = 1.0 / jnp.maximum(deg, 1.0)
    s0 = seg_ref[0] * inv[:, None]
    s1 = seg_ref[1] * inv[:, None]
    hn = h_ref[...] + s0 @ w23_ref[0] + s1 @ w23_ref[1]
    hn_ref[...] = hn
    p = hn @ wt_ref[...]
    pn_ref[0] = p[:, :128]
    pn_ref[1] = p[:, 128:]


def _update(h, seg, degpart, w23, w1t_next, interpret=False):
    return pl.pallas_call(
        _update_body,
        grid=(NP // BN,),
        in_specs=[
            pl.BlockSpec((BN, H), lambda i: (i, 0)),
            pl.BlockSpec((2, BN, 128), lambda i: (0, i, 0)),
            pl.BlockSpec((NSUB, BN), lambda i: (0, i)),
            pl.BlockSpec((2, 128, H), lambda i: (0, 0, 0)),
            pl.BlockSpec((H, H), lambda i: (0, 0)),
        ],
        out_specs=[
            pl.BlockSpec((BN, H), lambda i: (i, 0)),
            pl.BlockSpec((2, BN, 128), lambda i: (0, i, 0)),
        ],
        out_shape=[
            jax.ShapeDtypeStruct((NP, H), jnp.float32),
            jax.ShapeDtypeStruct((2, NP, 128), jnp.float32),
        ],
        interpret=interpret,
    )(h, seg, degpart, w23, w1t_next)


def _final_body(h_ref, seg_ref, dp_ref, w23_ref, b_ref, wout_ref, out_ref,
                acc, cnt):
    i = pl.program_id(0)

    @pl.when(i == 0)
    def _():
        acc[...] = jnp.zeros_like(acc)
        cnt[...] = jnp.zeros_like(cnt)

    deg = jnp.sum(dp_ref[...], axis=0)
    inv = 1.0 / jnp.maximum(deg, 1.0)
    s0 = seg_ref[0] * inv[:, None]
    s1 = seg_ref[1] * inv[:, None]
    hn = h_ref[...] + s0 @ w23_ref[0] + s1 @ w23_ref[1]

    b = b_ref[0]
    gids = lax.broadcasted_iota(jnp.int32, (G, BN), 0)
    onehot = (b[None, :] == gids).astype(jnp.float32)
    acc[...] += onehot @ hn
    cnt[...] += jnp.broadcast_to(
        jnp.sum(onehot, axis=1, keepdims=True), (G, 128))

    @pl.when(i == NP // BN - 1)
    def _():
        invc = 1.0 / jnp.maximum(cnt[...], 1.0)
        pooled = acc[...] * jnp.concatenate([invc, invc], axis=1)
        out_ref[...] = pooled @ wout_ref[...]


def _final(h, seg, degpart, w23, batch_p, w_out, interpret=False):
    return pl.pallas_call(
        _final_body,
        grid=(NP // BN,),
        in_specs=[
            pl.BlockSpec((BN, H), lambda i: (i, 0)),
            pl.BlockSpec((2, BN, 128), lambda i: (0, i, 0)),
            pl.BlockSpec((NSUB, BN), lambda i: (0, i)),
            pl.BlockSpec((2, 128, H), lambda i: (0, 0, 0)),
            pl.BlockSpec((1, BN), lambda i: (0, i)),
            pl.BlockSpec((H, OUT), lambda i: (0, 0)),
        ],
        out_specs=pl.BlockSpec((G, OUT), lambda i: (0, 0)),
        out_shape=jax.ShapeDtypeStruct((G, OUT), jnp.float32),
        scratch_shapes=[
            pltpu.VMEM((G, H), jnp.float32),
            pltpu.VMEM((G, 128), jnp.float32),
        ],
        interpret=interpret,
    )(h, seg, degpart, w23, batch_p, w_out)


@functools.lru_cache(maxsize=None)
def _mesh():
    return plsc.VectorSubcoreMesh(core_axis_name="c", subcore_axis_name="s")


NCH = EPT // C


@functools.lru_cache(maxsize=None)
def _make_edge_kernel(l, with_deg):
    seg_type = jax.ShapeDtypeStruct((2, NP, 128), jnp.float32)
    if with_deg:
        out_type = [seg_type, jax.ShapeDtypeStruct((NSUB, NP), jnp.float32)]
    else:
        out_type = seg_type
    scratch = [
        pltpu.VMEM((2, C), jnp.int32),
        pltpu.VMEM((2, C), jnp.int32),
        pltpu.VMEM((2, C), jnp.int32),
        pltpu.VMEM((2, C), jnp.int32),
        pltpu.VMEM((2, C, 128), jnp.float32),
        pltpu.VMEM((C, 64), jnp.uint32),
        pltpu.VMEM_SHARED((NP, 128), jnp.float32),
    ]
    if with_deg:
        scratch.append(pltpu.VMEM((NP,), jnp.float32))
    scratch += [pltpu.SemaphoreType.DMA] * 7

    def _edge_body(*refs):
        p_hbm, et_hbm, src_hbm, dst_hbm, seg_hbm = refs[:5]
        refs = refs[5:]
        if with_deg:
            deg_hbm = refs[0]
            refs = refs[1:]
        src_v, dst_v, gidx_v, sidx_v, rows_v, et_v, acc_sh = refs[:7]
        refs = refs[7:]
        if with_deg:
            hist_v = refs[0]
            refs = refs[1:]
        isem = refs[0:2]
        esem = refs[2]
        gsem = refs[3:5]
        ssem = refs[5:7]

        cc = lax.axis_index("c")
        ss = lax.axis_index("s")
        zeros16 = jnp.zeros((16,), jnp.float32)
        ones16 = jnp.ones((16,), jnp.float32)
        coff = cc * NP
        e_base = ss * EPT
        et_l = et_hbm.at[l].at[cc]

        rz = rows_v.at[0]

        def zrow(r, _):
            for j in range(8):
                rz[r, pl.ds(j * 16, 16)] = zeros16
            return 0

        lax.fori_loop(0, C, zrow, 0)
        for k in range(ROWS_PT // C):
            pltpu.sync_copy(rz, acc_sh.at[pl.ds(ss * ROWS_PT + k * C, C)])
        if with_deg:
            def zhist(j, _):
                hist_v[pl.ds(j * 16, 16)] = zeros16
                return 0

            lax.fori_loop(0, NP // 16, zhist, 0)

        def prep(b, kbase):
            for j in range(C // 16):
                sl = pl.ds(j * 16, 16)
                d = dst_v[b, sl]
                sidx_v[b, sl] = d
                gidx_v[b, sl] = src_v[b, sl] + coff
                if with_deg:
                    @pl.when(cc == 0)
                    def _():
                        plsc.addupdate_scatter(hist_v, [d], ones16)

        def start_idx(b, kbase):
            pltpu.async_copy(src_hbm.at[pl.ds(kbase, C)], src_v.at[b],
                             isem[b])
            pltpu.async_copy(dst_hbm.at[pl.ds(kbase, C)], dst_v.at[b],
                             isem[b])

        def wait_idx(b, kbase):
            pltpu.make_async_copy(src_hbm.at[pl.ds(kbase, C)], src_v.at[b],
                                  isem[b]).wait()
            pltpu.make_async_copy(dst_hbm.at[pl.ds(kbase, C)], dst_v.at[b],
                                  isem[b]).wait()

        def start_gather(b):
            pltpu.async_copy(p_hbm.at[gidx_v.at[b]], rows_v.at[b], gsem[b])

        def wait_gather(b):
            pltpu.make_async_copy(p_hbm.at[gidx_v.at[b]], rows_v.at[b],
                                  gsem[b]).wait()

        def start_et(kbase):
            pltpu.async_copy(et_l.at[pl.ds(kbase, C)], et_v, esem)

        def wait_et(kbase):
            pltpu.make_async_copy(et_l.at[pl.ds(kbase, C)], et_v,
                                  esem).wait()

        def start_scatter(b):
            pltpu.async_copy(rows_v.at[b], acc_sh.at[sidx_v.at[b]], ssem[b],
                             add=True)

        def wait_scatter(b):
            pltpu.make_async_copy(rows_v.at[b], acc_sh.at[sidx_v.at[b]],
                                  ssem[b]).wait()

        start_idx(0, e_base)
        start_et(e_base)
        start_idx(1, e_base + C)
        wait_idx(0, e_base)
        prep(0, e_base)
        start_gather(0)
        plsc.subcore_barrier()

        def chbody(ch, _):
            for b in (0, 1):
                nb = 1 - b
                k = 2 * ch + b
                kb = e_base + k * C

                if b == 0:
                    @pl.when(ch > 0)
                    def _():
                        wait_scatter(nb)
                else:
                    wait_scatter(nb)

                def launch_next():
                    wait_idx(nb, kb + C)
                    prep(nb, kb + C)
                    start_gather(nb)

                if b == 0:
                    launch_next()
                else:
                    @pl.when(ch < NCH // 2 - 1)
                    def _():
                        launch_next()

                @pl.when(ch < NCH // 2 - 1)
                def _():
                    start_idx(b, kb + 2 * C)

                wait_et(kb)
                wait_gather(b)
                rv = rows_v.at[b]

                def rbody(r, _):
                    for j in range(4):
                        w = et_v[r, pl.ds(j * 16, 16)]
                        ab = plsc.bitcast(w, jnp.bfloat16)
                        ea, eb = plsc.unpack(
                            ab, format=plsc.PackFormat.INTERLEAVED)
                        sa = pl.ds(j * 16, 16)
                        sb = pl.ds(64 + j * 16, 16)
                        rv[r, sa] = jnp.maximum(rv[r, sa] + ea, 0.0)
                        rv[r, sb] = jnp.maximum(rv[r, sb] + eb, 0.0)
                    return 0

                lax.fori_loop(0, C, rbody, 0)

                if b == 0:
                    start_et(kb + C)
                else:
                    @pl.when(ch < NCH // 2 - 1)
                    def _():
                        start_et(kb + C)

                start_scatter(b)
            return 0

        lax.fori_loop(0, NCH // 2, chbody, 0)
        wait_scatter(1)
        plsc.subcore_barrier()

        rs = rows_v.at[0]
        for k in range(ROWS_PT // C):
            base = ss * ROWS_PT + k * C
            pltpu.sync_copy(acc_sh.at[pl.ds(base, C)], rs)
            pltpu.sync_copy(rs, seg_hbm.at[cc].at[pl.ds(base, C)])
        if with_deg:
            @pl.when(cc == 0)
            def _():
                pltpu.sync_copy(hist_v, deg_hbm.at[ss])

    return functools.partial(
        pl.kernel,
        out_type=out_type,
        mesh=_mesh(),
        scratch_types=scratch,
        compiler_params=pltpu.CompilerParams(needs_layout_passes=False),
    )(_edge_body)


def kernel(x, edge_index, edge_features, batch, W_node, W_edge, W1, W2, W3,
           W_out):
    src = edge_index[0]
    dst = edge_index[1]
    xp = jnp.pad(x, ((0, NP - N), (0, 0)))
    batch_p = jnp.pad(batch, (0, NP - N), constant_values=G).reshape(1, NP)

    w1t = W1[:, :H, :]
    w1b = W1[:, H:, :]
    k = jnp.einsum("ih,lho->lio", W_edge, w1b)
    w23 = jnp.einsum("lhk,lko->lho", W2, W3).reshape(L, 2, 128, H)

    h, p = _node_init(xp, W_node, w1t[0])
    et0 = _eterm(edge_features, k[:1])
    et12 = _eterm(edge_features, k[1:])

    for l in range(L):
        if l == 0:
            seg, degpart = _make_edge_kernel(0, True)(
                p.reshape(2 * NP, 128), et0, src, dst)
        else:
            seg = _make_edge_kernel(l - 1, False)(
                p.reshape(2 * NP, 128), et12, src, dst)
        if l < L - 1:
            h, p = _update(h, seg, degpart, w23[l], w1t[l + 1])
        else:
            out = _final(h, seg, degpart, w23[l], batch_p, W_out)
    return out


# speedup vs baseline: 4.2062x; 1.0002x over previous
import functools

import jax
import jax.numpy as jnp
from jax import lax
from jax.experimental import pallas as pl
from jax.experimental.pallas import tpu as pltpu
from jax.experimental.pallas import tpu_sc as plsc

N = 10000
E = 320000
IN = 128
H = 256
OUT = 128
L = 3
G = 64

NP = 10240
BN = 1024
BE = 1000
C = 80
NSUB = 16
EPT = E // NSUB
ROWS_PT = NP // NSUB


def _node_init_body(x_ref, wn_ref, wt_ref, h_ref, p_ref):
    h = x_ref[...] @ wn_ref[...]
    h_ref[...] = h
    p = h @ wt_ref[...]
    p_ref[0] = p[:, :128]
    p_ref[1] = p[:, 128:]


def _node_init(xp, w_node, w1t0, interpret=False):
    return pl.pallas_call(
        _node_init_body,
        grid=(NP // BN,),
        in_specs=[
            pl.BlockSpec((BN, IN), lambda i: (i, 0)),
            pl.BlockSpec((IN, H), lambda i: (0, 0)),
            pl.BlockSpec((H, H), lambda i: (0, 0)),
        ],
        out_specs=[
            pl.BlockSpec((BN, H), lambda i: (i, 0)),
            pl.BlockSpec((2, BN, 128), lambda i: (0, i, 0)),
        ],
        out_shape=[
            jax.ShapeDtypeStruct((NP, H), jnp.float32),
            jax.ShapeDtypeStruct((2, NP, 128), jnp.float32),
        ],
        interpret=interpret,
    )(xp, w_node, w1t0)


def _pack_bf16_pairs(m):
    lo = jax.lax.bitcast_convert_type(
        m[:, :64].astype(jnp.bfloat16), jnp.uint16).astype(jnp.uint32)
    hi = jax.lax.bitcast_convert_type(
        m[:, 64:].astype(jnp.bfloat16), jnp.uint16).astype(jnp.uint32)
    return lo | (hi << 16)


def _make_eterm_body(nl):
    def _eterm_body(ef_ref, k_ref, out_ref):
        ef = ef_ref[...].astype(jnp.bfloat16)
        for l in range(nl):
            r = jax.lax.dot(ef, k_ref[l].astype(jnp.bfloat16),
                            preferred_element_type=jnp.float32)
            out_ref[l, 0] = _pack_bf16_pairs(r[:, :128])
            out_ref[l, 1] = _pack_bf16_pairs(r[:, 128:])
    return _eterm_body


def _eterm(edge_features, k, interpret=False):
    nl = k.shape[0]
    return pl.pallas_call(
        _make_eterm_body(nl),
        grid=(E // BE,),
        in_specs=[
            pl.BlockSpec((BE, IN), lambda i: (i, 0)),
            pl.BlockSpec((nl, IN, H), lambda i: (0, 0, 0)),
        ],
        out_specs=pl.BlockSpec((nl, 2, BE, 64), lambda i: (0, 0, i, 0)),
        out_shape=jax.ShapeDtypeStruct((nl, 2, E, 64), jnp.uint32),
        interpret=interpret,
    )(edge_features, k)


def _update_body(h_ref, seg_ref, dp_ref, w23_ref, wt_ref, hn_ref, pn_ref):
    deg = jnp.sum(dp_ref[...], axis=0)
    inv = 1.0 / jnp.maximum(deg, 1.0)
    s0 = seg_ref[0] * inv[:, None]
    s1 = seg_ref[1] * inv[:, None]
    hn = h_ref[...] + s0 @ w23_ref[0] + s1 @ w23_ref[1]
    hn_ref[...] = hn
    p = hn @ wt_ref[...]
    pn_ref[0] = p[:, :128]
    pn_ref[1] = p[:, 128:]


def _update(h, seg, degpart, w23, w1t_next, interpret=False):
    return pl.pallas_call(
        _update_body,
        grid=(NP // BN,),
        in_specs=[
            pl.BlockSpec((BN, H), lambda i: (i, 0)),
            pl.BlockSpec((2, BN, 128), lambda i: (0, i, 0)),
            pl.BlockSpec((NSUB, BN), lambda i: (0, i)),
            pl.BlockSpec((2, 128, H), lambda i: (0, 0, 0)),
            pl.BlockSpec((H, H), lambda i: (0, 0)),
        ],
        out_specs=[
            pl.BlockSpec((BN, H), lambda i: (i, 0)),
            pl.BlockSpec((2, BN, 128), lambda i: (0, i, 0)),
        ],
        out_shape=[
            jax.ShapeDtypeStruct((NP, H), jnp.float32),
            jax.ShapeDtypeStruct((2, NP, 128), jnp.float32),
        ],
        interpret=interpret,
    )(h, seg, degpart, w23, w1t_next)


def _final_body(h_ref, seg_ref, dp_ref, w23_ref, b_ref, wout_ref, out_ref,
                acc, cnt):
    i = pl.program_id(0)

    @pl.when(i == 0)
    def _():
        acc[...] = jnp.zeros_like(acc)
        cnt[...] = jnp.zeros_like(cnt)

    deg = jnp.sum(dp_ref[...], axis=0)
    inv = 1.0 / jnp.maximum(deg, 1.0)
    s0 = seg_ref[0] * inv[:, None]
    s1 = seg_ref[1] * inv[:, None]
    hn = h_ref[...] + s0 @ w23_ref[0] + s1 @ w23_ref[1]

    b = b_ref[0]
    gids = lax.broadcasted_iota(jnp.int32, (G, BN), 0)
    onehot = (b[None, :] == gids).astype(jnp.float32)
    acc[...] += onehot @ hn
    cnt[...] += jnp.broadcast_to(
        jnp.sum(onehot, axis=1, keepdims=True), (G, 128))

    @pl.when(i == NP // BN - 1)
    def _():
        invc = 1.0 / jnp.maximum(cnt[...], 1.0)
        pooled = acc[...] * jnp.concatenate([invc, invc], axis=1)
        out_ref[...] = pooled @ wout_ref[...]


def _final(h, seg, degpart, w23, batch_p, w_out, interpret=False):
    return pl.pallas_call(
        _final_body,
        grid=(NP // BN,),
        in_specs=[
            pl.BlockSpec((BN, H), lambda i: (i, 0)),
            pl.BlockSpec((2, BN, 128), lambda i: (0, i, 0)),
            pl.BlockSpec((NSUB, BN), lambda i: (0, i)),
            pl.BlockSpec((2, 128, H), lambda i: (0, 0, 0)),
            pl.BlockSpec((1, BN), lambda i: (0, i)),
            pl.BlockSpec((H, OUT), lambda i: (0, 0)),
        ],
        out_specs=pl.BlockSpec((G, OUT), lambda i: (0, 0)),
        out_shape=jax.ShapeDtypeStruct((G, OUT), jnp.float32),
        scratch_shapes=[
            pltpu.VMEM((G, H), jnp.float32),
            pltpu.VMEM((G, 128), jnp.float32),
        ],
        interpret=interpret,
    )(h, seg, degpart, w23, batch_p, w_out)


@functools.lru_cache(maxsize=None)
def _mesh():
    return plsc.VectorSubcoreMesh(core_axis_name="c", subcore_axis_name="s")


NCH = EPT // C


@functools.lru_cache(maxsize=None)
def _make_edge_kernel(l, with_deg):
    seg_type = jax.ShapeDtypeStruct((2, NP, 128), jnp.float32)
    if with_deg:
        out_type = [seg_type, jax.ShapeDtypeStruct((NSUB, NP), jnp.float32)]
    else:
        out_type = seg_type
    scratch = [
        pltpu.VMEM((2, C), jnp.int32),
        pltpu.VMEM((2, C), jnp.int32),
        pltpu.VMEM((2, C), jnp.int32),
        pltpu.VMEM((2, C), jnp.int32),
        pltpu.VMEM((2, C, 128), jnp.float32),
        pltpu.VMEM((C, 64), jnp.uint32),
        pltpu.VMEM_SHARED((NP, 128), jnp.float32),
    ]
    if with_deg:
        scratch.append(pltpu.VMEM((NP,), jnp.float32))
    scratch += [pltpu.SemaphoreType.DMA] * 7

    def _edge_body(*refs):
        p_hbm, et_hbm, src_hbm, dst_hbm, seg_hbm = refs[:5]
        refs = refs[5:]
        if with_deg:
            deg_hbm = refs[0]
            refs = refs[1:]
        src_v, dst_v, gidx_v, sidx_v, rows_v, et_v, acc_sh = refs[:7]
        refs = refs[7:]
        if with_deg:
            hist_v = refs[0]
            refs = refs[1:]
        isem = refs[0:2]
        esem = refs[2]
        gsem = refs[3:5]
        ssem = refs[5:7]

        cc = lax.axis_index("c")
        ss = lax.axis_index("s")
        zeros16 = jnp.zeros((16,), jnp.float32)
        ones16 = jnp.ones((16,), jnp.float32)
        coff = cc * NP
        e_base = ss * EPT
        et_l = et_hbm.at[l].at[cc]

        rz = rows_v.at[0]

        def zrow(r, _):
            for j in range(8):
                rz[r, pl.ds(j * 16, 16)] = zeros16
            return 0

        lax.fori_loop(0, C, zrow, 0)
        for k in range(ROWS_PT // C):
            pltpu.sync_copy(rz, acc_sh.at[pl.ds(ss * ROWS_PT + k * C, C)])
        if with_deg:
            def zhist(j, _):
                hist_v[pl.ds(j * 16, 16)] = zeros16
                return 0

            lax.fori_loop(0, NP // 16, zhist, 0)

        def prep(b, kbase):
            for j in range(C // 16):
                sl = pl.ds(j * 16, 16)
                d = dst_v[b, sl]
                sidx_v[b, sl] = d
                gidx_v[b, sl] = src_v[b, sl] + coff
                if with_deg:
                    @pl.when(cc == 0)
                    def _():
                        plsc.addupdate_scatter(hist_v, [d], ones16)

        def start_idx(b, kbase):
            pltpu.async_copy(src_hbm.at[pl.ds(kbase, C)], src_v.at[b],
                             isem[b])
            pltpu.async_copy(dst_hbm.at[pl.ds(kbase, C)], dst_v.at[b],
                             isem[b])

        def wait_idx(b, kbase):
            pltpu.make_async_copy(src_hbm.at[pl.ds(kbase, C)], src_v.at[b],
                                  isem[b]).wait()
            pltpu.make_async_copy(dst_hbm.at[pl.ds(kbase, C)], dst_v.at[b],
                                  isem[b]).wait()

        def start_gather(b):
            pltpu.async_copy(p_hbm.at[gidx_v.at[b]], rows_v.at[b], gsem[b])

        def wait_gather(b):
            pltpu.make_async_copy(p_hbm.at[gidx_v.at[b]], rows_v.at[b],
                                  gsem[b]).wait()

        def start_et(kbase):
            pltpu.async_copy(et_l.at[pl.ds(kbase, C)], et_v, esem)

        def wait_et(kbase):
            pltpu.make_async_copy(et_l.at[pl.ds(kbase, C)], et_v,
                                  esem).wait()

        def start_scatter(b):
            pltpu.async_copy(rows_v.at[b], acc_sh.at[sidx_v.at[b]], ssem[b],
                             add=True)

        def wait_scatter(b):
            pltpu.make_async_copy(rows_v.at[b], acc_sh.at[sidx_v.at[b]],
                                  ssem[b]).wait()

        start_idx(0, e_base)
        start_et(e_base)
        start_idx(1, e_base + C)
        wait_idx(0, e_base)
        prep(0, e_base)
        start_gather(0)
        plsc.subcore_barrier()

        def chbody(ch, _):
            for b in (0, 1):
                nb = 1 - b
                k = 2 * ch + b
                kb = e_base + k * C

                if b == 0:
                    @pl.when(ch > 0)
                    def _():
                        wait_scatter(nb)
                else:
                    wait_scatter(nb)

                def launch_next():
                    wait_idx(nb, kb + C)
                    prep(nb, kb + C)
                    start_gather(nb)

                if b == 0:
                    launch_next()
                else:
                    @pl.when(ch < NCH // 2 - 1)
                    def _():
                        launch_next()

                @pl.when(ch < NCH // 2 - 1)
                def _():
                    start_idx(b, kb + 2 * C)

                wait_et(kb)
                wait_gather(b)
                rv = rows_v.at[b]

                def rbody(r, _):
                    for j in range(4):
                        w = et_v[r, pl.ds(j * 16, 16)]
                        ab = plsc.bitcast(w, jnp.bfloat16)
                        ea, eb = plsc.unpack(
                            ab, format=plsc.PackFormat.INTERLEAVED)
                        sa = pl.ds(j * 16, 16)
                        sb = pl.ds(64 + j * 16, 16)
                        rv[r, sa] = jnp.maximum(rv[r, sa] + ea, 0.0)
                        rv[r, sb] = jnp.maximum(rv[r, sb] + eb, 0.0)
                    return 0

                lax.fori_loop(0, C, rbody, 0)

                if b == 0:
                    start_et(kb + C)
                else:
                    @pl.when(ch < NCH // 2 - 1)
                    def _():
                        start_et(kb + C)

                start_scatter(b)
            return 0

        lax.fori_loop(0, NCH // 2, chbody, 0)
        wait_scatter(1)
        plsc.subcore_barrier()

        rs = rows_v.at[0]
        for k in range(ROWS_PT // C):
            base = ss * ROWS_PT + k * C
            pltpu.sync_copy(acc_sh.at[pl.ds(base, C)], rs)
            pltpu.sync_copy(rs, seg_hbm.at[cc].at[pl.ds(base, C)])
        if with_deg:
            @pl.when(cc == 0)
            def _():
                pltpu.sync_copy(hist_v, deg_hbm.at[ss])

    return functools.partial(
        pl.kernel,
        out_type=out_type,
        mesh=_mesh(),
        scratch_types=scratch,
        compiler_params=pltpu.CompilerParams(needs_layout_passes=False),
    )(_edge_body)


def kernel(x, edge_index, edge_features, batch, W_node, W_edge, W1, W2, W3,
           W_out):
    src = edge_index[0]
    dst = edge_index[1]
    xp = jnp.pad(x, ((0, NP - N), (0, 0)))
    batch_p = jnp.pad(batch, (0, NP - N), constant_values=G).reshape(1, NP)

    w1t = W1[:, :H, :]
    w1b = W1[:, H:, :]
    k = jnp.einsum("ih,lho->lio", W_edge, w1b)
    w23 = jnp.einsum("lhk,lko->lho", W2, W3).reshape(L, 2, 128, H)

    h, p = _node_init(xp, W_node, w1t[0])
    et0 = _eterm(edge_features, k[:1])
    et12 = _eterm(edge_features, k[1:])

    for l in range(L):
        if l == 0:
            seg, degpart = _make_edge_kernel(0, True)(
                p.reshape(2 * NP, 128), et0, src, dst)
        else:
            seg = _make_edge_kernel(l - 1, False)(
                p.reshape(2 * NP, 128), et12, src, dst)
        if l < L - 1:
            h, p = _update(h, seg, degpart, w23[l], w1t[l + 1])
        else:
            out = _final(h, seg, degpart, w23[l], batch_p, W_out)
    return out


# speedup vs baseline: 4.4984x; 1.0695x over previous
import functools

import jax
import jax.numpy as jnp
from jax import lax
from jax.experimental import pallas as pl
from jax.experimental.pallas import tpu as pltpu
from jax.experimental.pallas import tpu_sc as plsc

N = 10000
E = 320000
IN = 128
H = 256
OUT = 128
L = 3
G = 64

NP = 10240
BN = 1024
BE = 4000
C = 80
NSUB = 16
EPT = E // NSUB
ROWS_PT = NP // NSUB


def _node_init_body(x_ref, wn_ref, wt_ref, h_ref, p_ref):
    h = x_ref[...] @ wn_ref[...]
    h_ref[...] = h
    p = h @ wt_ref[...]
    p_ref[0] = p[:, :128]
    p_ref[1] = p[:, 128:]


def _node_init(xp, w_node, w1t0, interpret=False):
    return pl.pallas_call(
        _node_init_body,
        grid=(NP // BN,),
        in_specs=[
            pl.BlockSpec((BN, IN), lambda i: (i, 0)),
            pl.BlockSpec((IN, H), lambda i: (0, 0)),
            pl.BlockSpec((H, H), lambda i: (0, 0)),
        ],
        out_specs=[
            pl.BlockSpec((BN, H), lambda i: (i, 0)),
            pl.BlockSpec((2, BN, 128), lambda i: (0, i, 0)),
        ],
        out_shape=[
            jax.ShapeDtypeStruct((NP, H), jnp.float32),
            jax.ShapeDtypeStruct((2, NP, 128), jnp.float32),
        ],
        interpret=interpret,
    )(xp, w_node, w1t0)


def _pack_bf16_pairs(m):
    lo = jax.lax.bitcast_convert_type(
        m[:, :64].astype(jnp.bfloat16), jnp.uint16).astype(jnp.uint32)
    hi = jax.lax.bitcast_convert_type(
        m[:, 64:].astype(jnp.bfloat16), jnp.uint16).astype(jnp.uint32)
    return lo | (hi << 16)


def _make_eterm_body(nl):
    def _eterm_body(ef_ref, k_ref, out_ref):
        ef = ef_ref[...].astype(jnp.bfloat16)
        for l in range(nl):
            r = jax.lax.dot(ef, k_ref[l].astype(jnp.bfloat16),
                            preferred_element_type=jnp.float32)
            out_ref[l, 0] = _pack_bf16_pairs(r[:, :128])
            out_ref[l, 1] = _pack_bf16_pairs(r[:, 128:])
    return _eterm_body


def _eterm(edge_features, k, interpret=False):
    nl = k.shape[0]
    return pl.pallas_call(
        _make_eterm_body(nl),
        grid=(E // BE,),
        in_specs=[
            pl.BlockSpec((BE, IN), lambda i: (i, 0)),
            pl.BlockSpec((nl, IN, H), lambda i: (0, 0, 0)),
        ],
        out_specs=pl.BlockSpec((nl, 2, BE, 64), lambda i: (0, 0, i, 0)),
        out_shape=jax.ShapeDtypeStruct((nl, 2, E, 64), jnp.uint32),
        interpret=interpret,
    )(edge_features, k)


def _update_body(h_ref, seg_ref, dp_ref, w23_ref, wt_ref, hn_ref, pn_ref):
    deg = jnp.sum(dp_ref[...], axis=0)
    inv = 1.0 / jnp.maximum(deg, 1.0)
    s0 = seg_ref[0] * inv[:, None]
    s1 = seg_ref[1] * inv[:, None]
    hn = h_ref[...] + s0 @ w23_ref[0] + s1 @ w23_ref[1]
    hn_ref[...] = hn
    p = hn @ wt_ref[...]
    pn_ref[0] = p[:, :128]
    pn_ref[1] = p[:, 128:]


def _update(h, seg, degpart, w23, w1t_next, interpret=False):
    return pl.pallas_call(
        _update_body,
        grid=(NP // BN,),
        in_specs=[
            pl.BlockSpec((BN, H), lambda i: (i, 0)),
            pl.BlockSpec((2, BN, 128), lambda i: (0, i, 0)),
            pl.BlockSpec((NSUB, BN), lambda i: (0, i)),
            pl.BlockSpec((2, 128, H), lambda i: (0, 0, 0)),
            pl.BlockSpec((H, H), lambda i: (0, 0)),
        ],
        out_specs=[
            pl.BlockSpec((BN, H), lambda i: (i, 0)),
            pl.BlockSpec((2, BN, 128), lambda i: (0, i, 0)),
        ],
        out_shape=[
            jax.ShapeDtypeStruct((NP, H), jnp.float32),
            jax.ShapeDtypeStruct((2, NP, 128), jnp.float32),
        ],
        interpret=interpret,
    )(h, seg, degpart, w23, w1t_next)


def _final_body(h_ref, seg_ref, dp_ref, w23_ref, b_ref, wout_ref, out_ref,
                acc, cnt):
    i = pl.program_id(0)

    @pl.when(i == 0)
    def _():
        acc[...] = jnp.zeros_like(acc)
        cnt[...] = jnp.zeros_like(cnt)

    deg = jnp.sum(dp_ref[...], axis=0)
    inv = 1.0 / jnp.maximum(deg, 1.0)
    s0 = seg_ref[0] * inv[:, None]
    s1 = seg_ref[1] * inv[:, None]
    hn = h_ref[...] + s0 @ w23_ref[0] + s1 @ w23_ref[1]

    b = b_ref[0]
    gids = lax.broadcasted_iota(jnp.int32, (G, BN), 0)
    onehot = (b[None, :] == gids).astype(jnp.float32)
    acc[...] += onehot @ hn
    cnt[...] += jnp.broadcast_to(
        jnp.sum(onehot, axis=1, keepdims=True), (G, 128))

    @pl.when(i == NP // BN - 1)
    def _():
        invc = 1.0 / jnp.maximum(cnt[...], 1.0)
        pooled = acc[...] * jnp.concatenate([invc, invc], axis=1)
        out_ref[...] = pooled @ wout_ref[...]


def _final(h, seg, degpart, w23, batch_p, w_out, interpret=False):
    return pl.pallas_call(
        _final_body,
        grid=(NP // BN,),
        in_specs=[
            pl.BlockSpec((BN, H), lambda i: (i, 0)),
            pl.BlockSpec((2, BN, 128), lambda i: (0, i, 0)),
            pl.BlockSpec((NSUB, BN), lambda i: (0, i)),
            pl.BlockSpec((2, 128, H), lambda i: (0, 0, 0)),
            pl.BlockSpec((1, BN), lambda i: (0, i)),
            pl.BlockSpec((H, OUT), lambda i: (0, 0)),
        ],
        out_specs=pl.BlockSpec((G, OUT), lambda i: (0, 0)),
        out_shape=jax.ShapeDtypeStruct((G, OUT), jnp.float32),
        scratch_shapes=[
            pltpu.VMEM((G, H), jnp.float32),
            pltpu.VMEM((G, 128), jnp.float32),
        ],
        interpret=interpret,
    )(h, seg, degpart, w23, batch_p, w_out)


@functools.lru_cache(maxsize=None)
def _mesh():
    return plsc.VectorSubcoreMesh(core_axis_name="c", subcore_axis_name="s")


NCH = EPT // C


@functools.lru_cache(maxsize=None)
def _make_edge_kernel(l, with_deg):
    seg_type = jax.ShapeDtypeStruct((2, NP, 128), jnp.float32)
    if with_deg:
        out_type = [seg_type, jax.ShapeDtypeStruct((NSUB, NP), jnp.float32)]
    else:
        out_type = seg_type
    scratch = [
        pltpu.VMEM((2, C), jnp.int32),
        pltpu.VMEM((2, C), jnp.int32),
        pltpu.VMEM((2, C), jnp.int32),
        pltpu.VMEM((2, C), jnp.int32),
        pltpu.VMEM((2, C, 128), jnp.float32),
        pltpu.VMEM((C, 64), jnp.uint32),
        pltpu.VMEM_SHARED((NP, 128), jnp.float32),
    ]
    if with_deg:
        scratch.append(pltpu.VMEM((NP,), jnp.float32))
    scratch += [pltpu.SemaphoreType.DMA] * 7

    def _edge_body(*refs):
        p_hbm, et_hbm, src_hbm, dst_hbm, seg_hbm = refs[:5]
        refs = refs[5:]
        if with_deg:
            deg_hbm = refs[0]
            refs = refs[1:]
        src_v, dst_v, gidx_v, sidx_v, rows_v, et_v, acc_sh = refs[:7]
        refs = refs[7:]
        if with_deg:
            hist_v = refs[0]
            refs = refs[1:]
        isem = refs[0:2]
        esem = refs[2]
        gsem = refs[3:5]
        ssem = refs[5:7]

        cc = lax.axis_index("c")
        ss = lax.axis_index("s")
        zeros16 = jnp.zeros((16,), jnp.float32)
        ones16 = jnp.ones((16,), jnp.float32)
        coff = cc * NP
        e_base = ss * EPT
        et_l = et_hbm.at[l].at[cc]

        rz = rows_v.at[0]

        def zrow(r, _):
            for j in range(8):
                rz[r, pl.ds(j * 16, 16)] = zeros16
            return 0

        lax.fori_loop(0, C, zrow, 0)
        for k in range(ROWS_PT // C):
            pltpu.sync_copy(rz, acc_sh.at[pl.ds(ss * ROWS_PT + k * C, C)])
        if with_deg:
            def zhist(j, _):
                hist_v[pl.ds(j * 16, 16)] = zeros16
                return 0

            lax.fori_loop(0, NP // 16, zhist, 0)

        def prep(b, kbase):
            for j in range(C // 16):
                sl = pl.ds(j * 16, 16)
                d = dst_v[b, sl]
                sidx_v[b, sl] = d
                gidx_v[b, sl] = src_v[b, sl] + coff
                if with_deg:
                    @pl.when(cc == 0)
                    def _():
                        plsc.addupdate_scatter(hist_v, [d], ones16)

        def start_idx(b, kbase):
            pltpu.async_copy(src_hbm.at[pl.ds(kbase, C)], src_v.at[b],
                             isem[b])
            pltpu.async_copy(dst_hbm.at[pl.ds(kbase, C)], dst_v.at[b],
                             isem[b])

        def wait_idx(b, kbase):
            pltpu.make_async_copy(src_hbm.at[pl.ds(kbase, C)], src_v.at[b],
                                  isem[b]).wait()
            pltpu.make_async_copy(dst_hbm.at[pl.ds(kbase, C)], dst_v.at[b],
                                  isem[b]).wait()

        def start_gather(b):
            pltpu.async_copy(p_hbm.at[gidx_v.at[b]], rows_v.at[b], gsem[b])

        def wait_gather(b):
            pltpu.make_async_copy(p_hbm.at[gidx_v.at[b]], rows_v.at[b],
                                  gsem[b]).wait()

        def start_et(kbase):
            pltpu.async_copy(et_l.at[pl.ds(kbase, C)], et_v, esem)

        def wait_et(kbase):
            pltpu.make_async_copy(et_l.at[pl.ds(kbase, C)], et_v,
                                  esem).wait()

        def start_scatter(b):
            pltpu.async_copy(rows_v.at[b], acc_sh.at[sidx_v.at[b]], ssem[b],
                             add=True)

        def wait_scatter(b):
            pltpu.make_async_copy(rows_v.at[b], acc_sh.at[sidx_v.at[b]],
                                  ssem[b]).wait()

        start_idx(0, e_base)
        start_et(e_base)
        start_idx(1, e_base + C)
        wait_idx(0, e_base)
        prep(0, e_base)
        start_gather(0)
        plsc.subcore_barrier()

        def chbody(ch, _):
            for b in (0, 1):
                nb = 1 - b
                k = 2 * ch + b
                kb = e_base + k * C

                if b == 0:
                    @pl.when(ch > 0)
                    def _():
                        wait_scatter(nb)
                else:
                    wait_scatter(nb)

                def launch_next():
                    wait_idx(nb, kb + C)
                    prep(nb, kb + C)
                    start_gather(nb)

                if b == 0:
                    launch_next()
                else:
                    @pl.when(ch < NCH // 2 - 1)
                    def _():
                        launch_next()

                @pl.when(ch < NCH // 2 - 1)
                def _():
                    start_idx(b, kb + 2 * C)

                wait_et(kb)
                wait_gather(b)
                rv = rows_v.at[b]

                def rbody(r, _):
                    for j in range(4):
                        w = et_v[r, pl.ds(j * 16, 16)]
                        ab = plsc.bitcast(w, jnp.bfloat16)
                        ea, eb = plsc.unpack(
                            ab, format=plsc.PackFormat.INTERLEAVED)
                        sa = pl.ds(j * 16, 16)
                        sb = pl.ds(64 + j * 16, 16)
                        rv[r, sa] = jnp.maximum(rv[r, sa] + ea, 0.0)
                        rv[r, sb] = jnp.maximum(rv[r, sb] + eb, 0.0)
                    return 0

                lax.fori_loop(0, C, rbody, 0)

                if b == 0:
                    start_et(kb + C)
                else:
                    @pl.when(ch < NCH // 2 - 1)
                    def _():
                        start_et(kb + C)

                start_scatter(b)
            return 0

        lax.fori_loop(0, NCH // 2, chbody, 0)
        wait_scatter(1)
        plsc.subcore_barrier()

        rs = rows_v.at[0]
        for k in range(ROWS_PT // C):
            base = ss * ROWS_PT + k * C
            pltpu.sync_copy(acc_sh.at[pl.ds(base, C)], rs)
            pltpu.sync_copy(rs, seg_hbm.at[cc].at[pl.ds(base, C)])
        if with_deg:
            @pl.when(cc == 0)
            def _():
                pltpu.sync_copy(hist_v, deg_hbm.at[ss])

    return functools.partial(
        pl.kernel,
        out_type=out_type,
        mesh=_mesh(),
        scratch_types=scratch,
        compiler_params=pltpu.CompilerParams(needs_layout_passes=False),
    )(_edge_body)


def kernel(x, edge_index, edge_features, batch, W_node, W_edge, W1, W2, W3,
           W_out):
    src = edge_index[0]
    dst = edge_index[1]
    xp = jnp.pad(x, ((0, NP - N), (0, 0)))
    batch_p = jnp.pad(batch, (0, NP - N), constant_values=G).reshape(1, NP)

    w1t = W1[:, :H, :]
    w1b = W1[:, H:, :]
    k = jnp.einsum("ih,lho->lio", W_edge, w1b)
    w23 = jnp.einsum("lhk,lko->lho", W2, W3).reshape(L, 2, 128, H)

    h, p = _node_init(xp, W_node, w1t[0])
    et0 = _eterm(edge_features, k[:1])
    et12 = _eterm(edge_features, k[1:])

    for l in range(L):
        if l == 0:
            seg, degpart = _make_edge_kernel(0, True)(
                p.reshape(2 * NP, 128), et0, src, dst)
        else:
            seg = _make_edge_kernel(l - 1, False)(
                p.reshape(2 * NP, 128), et12, src, dst)
        if l < L - 1:
            h, p = _update(h, seg, degpart, w23[l], w1t[l + 1])
        else:
            out = _final(h, seg, degpart, w23[l], batch_p, W_out)
    return out
